# baseline (speedup 1.0000x reference)
import jax
import jax.numpy as jnp
from jax import lax
from jax.experimental import pallas as pl
from jax.experimental.pallas import tpu as pltpu

_CAPACITY = 16384
_KEY_BITS = 2048
_VALUE_BITS = 2048
_BLK = 1024


def _body(q_ref, w_ref, qsum_ref, keys_ref, values_hbm, out_ref, best_ref, sem):
    i = pl.program_id(0)
    nblk = pl.num_programs(0)

    @pl.when(i == 0)
    def _init():
        best_ref[0] = jnp.int32(2**30)

    kb = keys_ref[...].astype(jnp.bfloat16)
    dot = jax.lax.dot_general(
        kb, w_ref[...],
        dimension_numbers=(((1,), (0,)), ((), ())),
        preferred_element_type=jnp.float32,
    )
    dist = qsum_ref[0] + dot.astype(jnp.int32)
    rows = lax.broadcasted_iota(jnp.int32, dist.shape, 0)
    combined = dist * _CAPACITY + (i * _BLK + rows)
    best_ref[0] = jnp.minimum(best_ref[0], jnp.min(combined))

    @pl.when(i == nblk - 1)
    def _gather():
        idx = jnp.bitwise_and(best_ref[0], _CAPACITY - 1)
        copy = pltpu.make_async_copy(values_hbm.at[idx], out_ref, sem)
        copy.start()
        copy.wait()


def kernel(query, keys, values):
    q2 = query.reshape(1, _KEY_BITS)
    w = (1 - 2 * query).astype(jnp.bfloat16).reshape(_KEY_BITS, 1)
    qsum = jnp.sum(query, dtype=jnp.int32).reshape(1)
    grid = _CAPACITY // _BLK
    return pl.pallas_call(
        _body,
        grid=(grid,),
        in_specs=[
            pl.BlockSpec((1, _KEY_BITS), lambda i: (0, 0)),
            pl.BlockSpec((_KEY_BITS, 1), lambda i: (0, 0)),
            pl.BlockSpec(memory_space=pltpu.SMEM),
            pl.BlockSpec((_BLK, _KEY_BITS), lambda i: (i, 0)),
            pl.BlockSpec(memory_space=pltpu.MemorySpace.HBM),
        ],
        out_specs=pl.BlockSpec(memory_space=pltpu.VMEM),
        out_shape=jax.ShapeDtypeStruct((_VALUE_BITS,), jnp.float32),
        scratch_shapes=[
            pltpu.SMEM((1,), jnp.int32),
            pltpu.SemaphoreType.DMA,
        ],
    )(q2, w, qsum, keys, values)


# speedup vs baseline: 1.1422x; 1.1422x over previous
import jax
import jax.numpy as jnp
from jax import lax
from jax.experimental import pallas as pl
from jax.experimental.pallas import tpu as pltpu

_CAPACITY = 16384
_KEY_BITS = 2048
_VALUE_BITS = 2048
_BLK = 1024


def _body(q_ref, keys_ref, values_hbm, out_ref, best_ref, sem):
    i = pl.program_id(0)
    nblk = pl.num_programs(0)

    @pl.when(i == 0)
    def _init():
        best_ref[0] = jnp.int32(2**30)

    xor = jnp.bitwise_xor(keys_ref[...], q_ref[...])
    dist = jnp.sum(xor, axis=1, keepdims=True)
    rows = lax.broadcasted_iota(jnp.int32, dist.shape, 0)
    combined = dist * _CAPACITY + (i * _BLK + rows)
    best_ref[0] = jnp.minimum(best_ref[0], jnp.min(combined))

    @pl.when(i == nblk - 1)
    def _gather():
        idx = jnp.bitwise_and(best_ref[0], _CAPACITY - 1)
        copy = pltpu.make_async_copy(values_hbm.at[idx], out_ref, sem)
        copy.start()
        copy.wait()


def kernel(query, keys, values):
    q2 = query.reshape(1, _KEY_BITS)
    grid = _CAPACITY // _BLK
    return pl.pallas_call(
        _body,
        grid=(grid,),
        in_specs=[
            pl.BlockSpec((1, _KEY_BITS), lambda i: (0, 0)),
            pl.BlockSpec((_BLK, _KEY_BITS), lambda i: (i, 0)),
            pl.BlockSpec(memory_space=pltpu.MemorySpace.HBM),
        ],
        out_specs=pl.BlockSpec(memory_space=pltpu.VMEM),
        out_shape=jax.ShapeDtypeStruct((_VALUE_BITS,), jnp.float32),
        scratch_shapes=[
            pltpu.SMEM((1,), jnp.int32),
            pltpu.SemaphoreType.DMA,
        ],
    )(q2, keys, values)
